# baseline (speedup 1.0000x reference)
import functools

import jax
import jax.numpy as jnp
from jax.experimental import pallas as pl
from jax.experimental.pallas import tpu as pltpu

_C_BLOCK = 32
_N_BUFFERS = 4


def _inner(idx, x_ref, wc_ref, bc_ref, wr_ref, br_ref, wd_ref, bd_ref,
           oc_ref, og_ref, od_ref):
    c = idx[1]
    x = x_ref[0]
    pc = jnp.dot(wc_ref[0], x, preferred_element_type=jnp.float32)
    pg = jnp.dot(wr_ref[0], x, preferred_element_type=jnp.float32)
    pd = jnp.dot(wd_ref[0], x, preferred_element_type=jnp.float32)

    @pl.when(c == 0)
    def _init():
        oc_ref[0] = pc + bc_ref[:]
        og_ref[0] = pg + br_ref[:]
        od_ref[0] = pd + bd_ref[:]

    @pl.when(c != 0)
    def _accum():
        oc_ref[0] += pc
        og_ref[0] += pg
        od_ref[0] += pd


def _outer(B, n_c, HW, Oc, Og, Od,
           xf_ref, wc_ref, bc_ref, wr_ref, br_ref, wd_ref, bd_ref,
           oc_ref, og_ref, od_ref):
    x_spec = pl.BlockSpec(
        (1, _C_BLOCK, HW), lambda b, c: (b, c, 0),
        pipeline_mode=pl.Buffered(buffer_count=_N_BUFFERS, use_lookahead=True))

    def w_spec(o):
        return pl.BlockSpec((1, o, _C_BLOCK), lambda b, c: (c, 0, 0))

    def b_spec(o):
        return pl.BlockSpec((o, 1), lambda b, c: (0, 0))

    def o_spec(o):
        return pl.BlockSpec((1, o, HW), lambda b, c: (b, 0, 0))

    pipe = pltpu.emit_pipeline(
        _inner,
        grid=(B, n_c),
        in_specs=[x_spec, w_spec(Oc), b_spec(Oc), w_spec(Og), b_spec(Og),
                  w_spec(Od), b_spec(Od)],
        out_specs=[o_spec(Oc), o_spec(Og), o_spec(Od)],
        dimension_semantics=(pltpu.PARALLEL, pltpu.ARBITRARY),
        _explicit_indices=True,
    )
    pipe(xf_ref, wc_ref, bc_ref, wr_ref, br_ref, wd_ref, bd_ref,
         oc_ref, og_ref, od_ref)


@jax.jit
def kernel(x, W_cls, b_cls, W_reg, b_reg, W_dir, b_dir):
    B, C, H, W = x.shape
    HW = H * W
    Oc = W_cls.shape[0]
    Og = W_reg.shape[0]
    Od = W_dir.shape[0]
    xf = x.reshape(B, C, HW)
    n_c = C // _C_BLOCK

    def w_split(w):
        o = w.shape[0]
        return w.reshape(o, n_c, _C_BLOCK).transpose(1, 0, 2)

    out_cls, out_reg, out_dir = pl.pallas_call(
        functools.partial(_outer, B, n_c, HW, Oc, Og, Od),
        in_specs=[pl.BlockSpec(memory_space=pl.ANY)] * 7,
        out_specs=(pl.BlockSpec(memory_space=pl.ANY),) * 3,
        out_shape=(
            jax.ShapeDtypeStruct((B, Oc, HW), jnp.float32),
            jax.ShapeDtypeStruct((B, Og, HW), jnp.float32),
            jax.ShapeDtypeStruct((B, Od, HW), jnp.float32),
        ),
    )(xf, w_split(W_cls), b_cls.reshape(Oc, 1), w_split(W_reg),
      b_reg.reshape(Og, 1), w_split(W_dir), b_dir.reshape(Od, 1))

    return (out_cls.reshape(B, Oc, H, W),
            out_reg.reshape(B, Og, H, W),
            out_dir.reshape(B, Od, H, W))


# speedup vs baseline: 1.0644x; 1.0644x over previous
import jax
import jax.numpy as jnp
from jax.experimental import pallas as pl
from jax.experimental.pallas import tpu as pltpu

_TILE_N = 8192
_N_STREAMS = 4


def _head_kernel(*refs):
    x_refs = refs[:_N_STREAMS]
    wc_ref, bc_ref, wr_ref, br_ref, wd_ref, bd_ref = refs[_N_STREAMS:-3]
    oc_ref, og_ref, od_ref = refs[-3:]
    cq = wc_ref.shape[1] // _N_STREAMS

    pc = pg = pd = None
    for i in range(_N_STREAMS):
        x = x_refs[i][0]
        sl = slice(i * cq, (i + 1) * cq)
        dc = jnp.dot(wc_ref[:, sl], x, preferred_element_type=jnp.float32)
        dg = jnp.dot(wr_ref[:, sl], x, preferred_element_type=jnp.float32)
        dd = jnp.dot(wd_ref[:, sl], x, preferred_element_type=jnp.float32)
        pc = dc if pc is None else pc + dc
        pg = dg if pg is None else pg + dg
        pd = dd if pd is None else pd + dd

    oc_ref[0] = pc + bc_ref[:]
    og_ref[0] = pg + br_ref[:]
    od_ref[0] = pd + bd_ref[:]


@jax.jit
def kernel(x, W_cls, b_cls, W_reg, b_reg, W_dir, b_dir):
    B, C, H, W = x.shape
    HW = H * W
    Oc = W_cls.shape[0]
    Og = W_reg.shape[0]
    Od = W_dir.shape[0]
    xf = x.reshape(B, C, HW)
    n_tiles = pl.cdiv(HW, _TILE_N)
    cq = C // _N_STREAMS

    def x_spec(i):
        return pl.BlockSpec((1, cq, _TILE_N), lambda b, n, i=i: (b, i, n))

    def w_spec(o):
        return pl.BlockSpec((o, C), lambda b, n: (0, 0))

    def b_spec(o):
        return pl.BlockSpec((o, 1), lambda b, n: (0, 0))

    def o_spec(o):
        return pl.BlockSpec((1, o, _TILE_N), lambda b, n: (b, 0, n))

    out_cls, out_reg, out_dir = pl.pallas_call(
        _head_kernel,
        grid=(B, n_tiles),
        in_specs=[x_spec(i) for i in range(_N_STREAMS)] + [
            w_spec(Oc), b_spec(Oc),
            w_spec(Og), b_spec(Og),
            w_spec(Od), b_spec(Od),
        ],
        out_specs=(o_spec(Oc), o_spec(Og), o_spec(Od)),
        out_shape=(
            jax.ShapeDtypeStruct((B, Oc, HW), jnp.float32),
            jax.ShapeDtypeStruct((B, Og, HW), jnp.float32),
            jax.ShapeDtypeStruct((B, Od, HW), jnp.float32),
        ),
        compiler_params=pltpu.CompilerParams(
            dimension_semantics=("parallel", "parallel"),
        ),
    )(xf, xf, xf, xf, W_cls, b_cls.reshape(Oc, 1), W_reg,
      b_reg.reshape(Og, 1), W_dir, b_dir.reshape(Od, 1))

    return (out_cls.reshape(B, Oc, H, W),
            out_reg.reshape(B, Og, H, W),
            out_dir.reshape(B, Od, H, W))
